# Initial kernel scaffold; baseline (speedup 1.0000x reference)
import jax
import jax.numpy as jnp
from jax.experimental import pallas as pl


def kernel(x, W):
    raise NotImplementedError("write your pallas kernel here")


# speedup vs baseline: 1.4067x; 1.4067x over previous
import functools

import jax
import jax.numpy as jnp
from jax.experimental import pallas as pl
from jax.experimental.pallas import tpu as pltpu

BM = 2048
BN = 2048
BK = 512


def _matmul_zero_diag_kernel(x_ref, w_ref, o_ref):
    kk = pl.program_id(2)
    nj = pl.program_id(1)

    @pl.when(kk == 0)
    def _():
        o_ref[...] = jnp.zeros_like(o_ref)

    w = w_ref[...]
    row_ids = kk * BK + jax.lax.broadcasted_iota(jnp.int32, (BK, BN), 0)
    col_ids = nj * BN + jax.lax.broadcasted_iota(jnp.int32, (BK, BN), 1)
    w = jnp.where(row_ids == col_ids, 0.0, w)
    o_ref[...] += jnp.dot(x_ref[...], w, preferred_element_type=jnp.float32)


@jax.jit
def kernel(x, W):
    M, K = x.shape
    K2, N = W.shape
    grid = (M // BM, N // BN, K // BK)
    return pl.pallas_call(
        _matmul_zero_diag_kernel,
        grid=grid,
        in_specs=[
            pl.BlockSpec((BM, BK), lambda mi, nj, kk: (mi, kk)),
            pl.BlockSpec((BK, BN), lambda mi, nj, kk: (kk, nj)),
        ],
        out_specs=pl.BlockSpec((BM, BN), lambda mi, nj, kk: (mi, nj)),
        out_shape=jax.ShapeDtypeStruct((M, N), jnp.float32),
        compiler_params=pltpu.CompilerParams(
            dimension_semantics=("parallel", "parallel", "arbitrary"),
        ),
    )(x, W)
